# Initial kernel scaffold; baseline (speedup 1.0000x reference)
import jax
import jax.numpy as jnp
from jax.experimental import pallas as pl


def kernel(feat0, feat1, feat2, feat3, feat4, boxes0, boxes1):
    raise NotImplementedError("write your pallas kernel here")


# speedup vs baseline: 13.9573x; 13.9573x over previous
import functools

import jax
import jax.numpy as jnp
import numpy as np
from jax import lax
from jax.experimental import pallas as pl
from jax.experimental.pallas import tpu as pltpu
from jax.experimental.pallas import tpu_sc as plsc

_POOL = 7
_SR = 2
_NS = _POOL * _SR
_C = 256
_K_PAD = 512
_ROIS_PER_W = 16
_NW = 32
_E = _NS * 2 * _NS * 2
_GCH = 112
_NCH = _E // _GCH
_SIZES = (200, 100, 50, 25)
_BASES = (0, 80000, 100000, 105000)
_SCALES = (0.25, 0.125, 0.0625, 0.03125)
_TABLE_ROWS = 106250


def _lane_tables():
    e = np.arange(_E)
    cx = e % 2
    sx = (e // 2) % _NS
    cy = (e // (2 * _NS)) % 2
    sy = e // (4 * _NS)
    ylane = cy * 16 + sy
    xlane = 32 + cx * 16 + sx
    binid = (sy // 2) * _POOL + (sx // 2)
    return (ylane.astype(np.int32), xlane.astype(np.int32),
            binid.astype(np.int32))


_YLANE, _XLANE, _BINID = _lane_tables()


def _meta_body(boxes_ref, mi_ref, mf_ref):
    bx = boxes_ref[...]
    x1 = bx[:, 0:1]
    y1 = bx[:, 1:2]
    x2 = bx[:, 2:3]
    y2 = bx[:, 3:4]
    row = lax.broadcasted_iota(jnp.int32, (_K_PAD, 1), 0)
    bid = jnp.where(row >= 250, 1.0, 0.0)
    area = (x2 - x1) * (y2 - y1)
    s = jnp.sqrt(area)
    t = jnp.floor(4.0 + jnp.log2(s / 224.0) + 1e-6)
    lvl = jnp.clip(t, 2.0, 5.0) - 2.0

    def sel(vals):
        r = jnp.full_like(lvl, vals[3])
        for i in (2, 1, 0):
            r = jnp.where(lvl <= i + 0.5, vals[i], r)
        return r

    scale = sel(_SCALES)
    hf = sel(tuple(float(v) for v in _SIZES))
    base = sel(tuple(float(v) for v in _BASES))
    rowbase = base + bid * hf * hf

    colmask = (lax.broadcasted_iota(jnp.int32, (_K_PAD, 16), 1)
               < _NS).astype(jnp.float32)
    coef = (lax.broadcasted_iota(jnp.int32, (_K_PAD, 16), 1)
            .astype(jnp.float32) * 0.5 + 0.25)
    valid_roi = jnp.where(row < 500, 1.0, 0.0)

    def side(lo, hi):
        los = lo * scale
        bsz = jnp.maximum(hi * scale - los, 1.0) / float(_POOL)
        cs = los + bsz * coef
        v = ((cs >= -1.0) & (cs <= hf)).astype(jnp.float32)
        c = jnp.maximum(cs, 0.0)
        cl = jnp.minimum(jnp.floor(c), hf - 1.0)
        ch = jnp.minimum(cl + 1.0, hf - 1.0)
        lw = jnp.where(c >= hf - 1.0, 0.0, c - cl)
        wl = (1.0 - lw) * v * 0.5 * colmask
        wh = lw * v * 0.5 * colmask
        return cl, ch, wl, wh

    yl, yh, wyl, wyh = side(y1, y2)
    xl, xh, wxl, wxh = side(x1, x2)
    mi_ref[:, 0, :] = (yl * hf).astype(jnp.int32)
    mi_ref[:, 1, :] = (yh * hf).astype(jnp.int32)
    mi_ref[:, 2, :] = (xl + rowbase).astype(jnp.int32)
    mi_ref[:, 3, :] = (xh + rowbase).astype(jnp.int32)
    mf_ref[:, 0, :] = wyl
    mf_ref[:, 1, :] = wyh
    mf_ref[:, 2, :] = wxl * valid_roi
    mf_ref[:, 3, :] = wxh * valid_roi


def _run_meta(boxes_padded):
    return pl.pallas_call(
        _meta_body,
        out_shape=(jax.ShapeDtypeStruct((_K_PAD, 4, 16), jnp.int32),
                   jax.ShapeDtypeStruct((_K_PAD, 4, 16), jnp.float32)),
    )(boxes_padded)


def _sc_body(table_hbm, mi_hbm, mf_hbm, ylane_hbm, xlane_hbm, bin_hbm,
             out_hbm,
             ylane_v, xlane_v, binv_v, mi_v, mf_v, idx_v, wv_v,
             buf0, buf1, acc_v, sem0, sem1):
    wid = lax.axis_index("s") * 2 + lax.axis_index("c")
    pltpu.sync_copy(ylane_hbm, ylane_v)
    pltpu.sync_copy(xlane_hbm, xlane_v)
    pltpu.sync_copy(bin_hbm, binv_v)
    zeros16 = jnp.zeros((16,), jnp.float32)
    bufs = (buf0, buf1)
    sems = (sem0, sem1)

    def roi_body(r, carry):
        roi = wid * _ROIS_PER_W + r
        pltpu.sync_copy(mi_hbm.at[pl.ds(roi * 64, 64)], mi_v)
        pltpu.sync_copy(mf_hbm.at[pl.ds(roi * 64, 64)], mf_v)

        def build_row(rr, carry2):
            def build_q(q, carry3):
                cb = rr * _NCH + q
                off = cb * 16
                yi = ylane_v[pl.ds(off, 16)]
                xi = xlane_v[pl.ds(off, 16)]
                ygw = plsc.load_gather(mi_v, [yi])
                xrb = plsc.load_gather(mi_v, [xi])
                wy = plsc.load_gather(mf_v, [yi])
                wx = plsc.load_gather(mf_v, [xi])
                idx_v[rr, pl.ds(q * 16, 16)] = ygw + xrb
                wv_v[pl.ds(off, 16)] = wy * wx
                return carry3
            return lax.fori_loop(0, _NCH, build_q, carry2)
        lax.fori_loop(0, _NCH, build_row, 0)

        def zero_bin(bz, carry2):
            for cc in range(16):
                acc_v[bz, pl.ds(cc * 16, 16)] = zeros16
            return carry2
        lax.fori_loop(0, _POOL * _POOL, zero_bin, 0)

        copies = []
        for c in range(_NCH):
            copies.append(pltpu.make_async_copy(
                table_hbm.at[idx_v.at[c]], bufs[c % 2], sems[c % 2]))
        copies[0].start()
        for c in range(_NCH):
            copies[c].wait()
            if c + 1 < _NCH:
                copies[c + 1].start()
            buf = bufs[c % 2]

            def process(g, carry2):
                off = c * _GCH + g * 16
                w16 = wv_v[pl.ds(off, 16)]
                b16 = binv_v[pl.ds(off, 16)]
                for j in range(16):
                    el = g * 16 + j
                    w_ = w16[j]
                    bin_ = b16[j]
                    for cc in range(16):
                        sl = pl.ds(cc * 16, 16)
                        plsc.addupdate(acc_v.at[bin_, sl], buf[el, sl] * w_)
                return carry2
            lax.fori_loop(0, _GCH // 16, process, 0)

        pltpu.sync_copy(acc_v, out_hbm.at[roi])
        return carry
    lax.fori_loop(0, _ROIS_PER_W, roi_body, 0)


def _run_sc(table, mi, mf, ylane, xlane, bins):
    mesh = plsc.VectorSubcoreMesh(core_axis_name="c", subcore_axis_name="s")
    kfn = functools.partial(
        pl.kernel,
        out_type=jax.ShapeDtypeStruct((_K_PAD, _POOL * _POOL, _C),
                                      jnp.float32),
        mesh=mesh,
        compiler_params=pltpu.CompilerParams(needs_layout_passes=False),
        scratch_types=[
            pltpu.VMEM((_E,), jnp.int32),
            pltpu.VMEM((_E,), jnp.int32),
            pltpu.VMEM((_E,), jnp.int32),
            pltpu.VMEM((64,), jnp.int32),
            pltpu.VMEM((64,), jnp.float32),
            pltpu.VMEM((_NCH, _GCH), jnp.int32),
            pltpu.VMEM((_E,), jnp.float32),
            pltpu.VMEM((_GCH, _C), jnp.float32),
            pltpu.VMEM((_GCH, _C), jnp.float32),
            pltpu.VMEM((_POOL * _POOL, _C), jnp.float32),
            pltpu.SemaphoreType.DMA,
            pltpu.SemaphoreType.DMA,
        ],
    )(_sc_body)
    return kfn(table, mi, mf, ylane, xlane, bins)


def kernel(feat0, feat1, feat2, feat3, feat4, boxes0, boxes1):
    del feat4
    feats = (feat0, feat1, feat2, feat3)
    table = jnp.concatenate(
        [jnp.transpose(f, (0, 2, 3, 1)).reshape(-1, _C) for f in feats],
        axis=0)
    pad = jnp.tile(jnp.array([[0.0, 0.0, 8.0, 8.0]], jnp.float32),
                   (_K_PAD - 500, 1))
    boxes = jnp.concatenate([boxes0, boxes1, pad], axis=0)
    mi, mf = _run_meta(boxes)
    out = _run_sc(table, mi.reshape(-1), mf.reshape(-1),
                  jnp.asarray(_YLANE), jnp.asarray(_XLANE),
                  jnp.asarray(_BINID))
    return jnp.transpose(out[:500], (0, 2, 1)).reshape(
        500, _C, _POOL, _POOL)


# speedup vs baseline: 26.0638x; 1.8674x over previous
import functools

import jax
import jax.numpy as jnp
import numpy as np
from jax import lax
from jax.experimental import pallas as pl
from jax.experimental.pallas import tpu as pltpu
from jax.experimental.pallas import tpu_sc as plsc

_POOL = 7
_SR = 2
_NS = _POOL * _SR
_C = 256
_K_PAD = 512
_ROIS_PER_W = 16
_NW = 32
_E = _NS * 2 * _NS * 2
_GCH = 112
_NCH = _E // _GCH
_SIZES = (200, 100, 50, 25)
_BASES = (0, 80000, 100000, 105000)
_SCALES = (0.25, 0.125, 0.0625, 0.03125)
_TABLE_ROWS = 106250


def _lane_tables():
    e = np.arange(_E)
    j = e % 16
    px = (e // 16) % _POOL
    py = e // (16 * _POOL)
    cx = j % 2
    ix = (j // 2) % 2
    cy = (j // 4) % 2
    iy = j // 8
    sy = py * _SR + iy
    sx = px * _SR + ix
    ylane = cy * 16 + sy
    xlane = 32 + cx * 16 + sx
    return ylane.astype(np.int32), xlane.astype(np.int32)


_YLANE, _XLANE = _lane_tables()


def _meta_body(boxes_ref, mi_ref, mf_ref):
    bx = boxes_ref[...]
    x1 = bx[:, 0:1]
    y1 = bx[:, 1:2]
    x2 = bx[:, 2:3]
    y2 = bx[:, 3:4]
    row = lax.broadcasted_iota(jnp.int32, (_K_PAD, 1), 0)
    bid = jnp.where(row >= 250, 1.0, 0.0)
    area = (x2 - x1) * (y2 - y1)
    s = jnp.sqrt(area)
    t = jnp.floor(4.0 + jnp.log2(s / 224.0) + 1e-6)
    lvl = jnp.clip(t, 2.0, 5.0) - 2.0

    def sel(vals):
        r = jnp.full_like(lvl, vals[3])
        for i in (2, 1, 0):
            r = jnp.where(lvl <= i + 0.5, vals[i], r)
        return r

    scale = sel(_SCALES)
    hf = sel(tuple(float(v) for v in _SIZES))
    base = sel(tuple(float(v) for v in _BASES))
    rowbase = base + bid * hf * hf

    colmask = (lax.broadcasted_iota(jnp.int32, (_K_PAD, 16), 1)
               < _NS).astype(jnp.float32)
    coef = (lax.broadcasted_iota(jnp.int32, (_K_PAD, 16), 1)
            .astype(jnp.float32) * 0.5 + 0.25)
    valid_roi = jnp.where(row < 500, 1.0, 0.0)

    def side(lo, hi):
        los = lo * scale
        bsz = jnp.maximum(hi * scale - los, 1.0) / float(_POOL)
        cs = los + bsz * coef
        v = ((cs >= -1.0) & (cs <= hf)).astype(jnp.float32)
        c = jnp.maximum(cs, 0.0)
        cl = jnp.minimum(jnp.floor(c), hf - 1.0)
        ch = jnp.minimum(cl + 1.0, hf - 1.0)
        lw = jnp.where(c >= hf - 1.0, 0.0, c - cl)
        wl = (1.0 - lw) * v * 0.5 * colmask
        wh = lw * v * 0.5 * colmask
        return cl, ch, wl, wh

    yl, yh, wyl, wyh = side(y1, y2)
    xl, xh, wxl, wxh = side(x1, x2)
    mi_ref[:, 0, :] = (yl * hf).astype(jnp.int32)
    mi_ref[:, 1, :] = (yh * hf).astype(jnp.int32)
    mi_ref[:, 2, :] = (xl + rowbase).astype(jnp.int32)
    mi_ref[:, 3, :] = (xh + rowbase).astype(jnp.int32)
    mf_ref[:, 0, :] = wyl
    mf_ref[:, 1, :] = wyh
    mf_ref[:, 2, :] = wxl * valid_roi
    mf_ref[:, 3, :] = wxh * valid_roi


def _run_meta(boxes_padded):
    return pl.pallas_call(
        _meta_body,
        out_shape=(jax.ShapeDtypeStruct((_K_PAD, 4, 16), jnp.int32),
                   jax.ShapeDtypeStruct((_K_PAD, 4, 16), jnp.float32)),
    )(boxes_padded)


def _sc_body(table_hbm, mi_hbm, mf_hbm, ylane_hbm, xlane_hbm,
             out_hbm,
             ylane_v, xlane_v, mi_v, mf_v, idx_v, wv_v,
             buf0, buf1, acc_v, sem0, sem1):
    wid = lax.axis_index("s") * 2 + lax.axis_index("c")
    pltpu.sync_copy(ylane_hbm, ylane_v)
    pltpu.sync_copy(xlane_hbm, xlane_v)
    bufs = (buf0, buf1)
    sems = (sem0, sem1)

    def roi_body(r, carry):
        roi = wid * _ROIS_PER_W + r
        pltpu.sync_copy(mi_hbm.at[pl.ds(roi * 64, 64)], mi_v)
        pltpu.sync_copy(mf_hbm.at[pl.ds(roi * 64, 64)], mf_v)

        def build_row(rr, carry2):
            def build_q(q, carry3):
                cb = rr * _NCH + q
                off = cb * 16
                yi = ylane_v[pl.ds(off, 16)]
                xi = xlane_v[pl.ds(off, 16)]
                ygw = plsc.load_gather(mi_v, [yi])
                xrb = plsc.load_gather(mi_v, [xi])
                wy = plsc.load_gather(mf_v, [yi])
                wx = plsc.load_gather(mf_v, [xi])
                idx_v[rr, pl.ds(q * 16, 16)] = ygw + xrb
                wv_v[pl.ds(off, 16)] = wy * wx
                return carry3
            return lax.fori_loop(0, _NCH, build_q, carry2)
        lax.fori_loop(0, _NCH, build_row, 0)

        copies = []
        for c in range(_NCH):
            copies.append(pltpu.make_async_copy(
                table_hbm.at[idx_v.at[c]], bufs[c % 2], sems[c % 2]))
        copies[0].start()
        for c in range(_NCH):
            copies[c].wait()
            if c + 1 < _NCH:
                copies[c + 1].start()
            buf = bufs[c % 2]

            def process_bin(b7, carry2):
                w16 = wv_v[pl.ds(c * _GCH + b7 * 16, 16)]
                wvecs = [jnp.broadcast_to(w16[j], (16,)) for j in range(16)]
                rowb = b7 * 16
                binrow = c * _POOL + b7
                for cc in range(16):
                    sl = pl.ds(cc * 16, 16)
                    ps = [buf[rowb + j, sl] * wvecs[j] for j in range(16)]
                    while len(ps) > 1:
                        ps = [ps[2 * i] + ps[2 * i + 1]
                              for i in range(len(ps) // 2)]
                    acc_v[binrow, sl] = ps[0]
                return carry2
            lax.fori_loop(0, _POOL, process_bin, 0)

        pltpu.sync_copy(acc_v, out_hbm.at[roi])
        return carry
    lax.fori_loop(0, _ROIS_PER_W, roi_body, 0)


def _run_sc(table, mi, mf, ylane, xlane):
    mesh = plsc.VectorSubcoreMesh(core_axis_name="c", subcore_axis_name="s")
    kfn = functools.partial(
        pl.kernel,
        out_type=jax.ShapeDtypeStruct((_K_PAD, _POOL * _POOL, _C),
                                      jnp.float32),
        mesh=mesh,
        compiler_params=pltpu.CompilerParams(needs_layout_passes=False),
        scratch_types=[
            pltpu.VMEM((_E,), jnp.int32),
            pltpu.VMEM((_E,), jnp.int32),
            pltpu.VMEM((64,), jnp.int32),
            pltpu.VMEM((64,), jnp.float32),
            pltpu.VMEM((_NCH, _GCH), jnp.int32),
            pltpu.VMEM((_E,), jnp.float32),
            pltpu.VMEM((_GCH, _C), jnp.float32),
            pltpu.VMEM((_GCH, _C), jnp.float32),
            pltpu.VMEM((_POOL * _POOL, _C), jnp.float32),
            pltpu.SemaphoreType.DMA,
            pltpu.SemaphoreType.DMA,
        ],
    )(_sc_body)
    return kfn(table, mi, mf, ylane, xlane)


def kernel(feat0, feat1, feat2, feat3, feat4, boxes0, boxes1):
    del feat4
    feats = (feat0, feat1, feat2, feat3)
    table = jnp.concatenate(
        [jnp.transpose(f, (0, 2, 3, 1)).reshape(-1, _C) for f in feats],
        axis=0)
    pad = jnp.tile(jnp.array([[0.0, 0.0, 8.0, 8.0]], jnp.float32),
                   (_K_PAD - 500, 1))
    boxes = jnp.concatenate([boxes0, boxes1, pad], axis=0)
    mi, mf = _run_meta(boxes)
    out = _run_sc(table, mi.reshape(-1), mf.reshape(-1),
                  jnp.asarray(_YLANE), jnp.asarray(_XLANE))
    return jnp.transpose(out[:500], (0, 2, 1)).reshape(
        500, _C, _POOL, _POOL)
